# baseline (speedup 1.0000x reference)
import dataclasses
import functools

import jax
import jax.numpy as jnp
from jax import lax
from jax.experimental import pallas as pl
from jax.experimental.pallas import tpu as pltpu
from jax.experimental.pallas import tpu_sc as plsc

B = 2
S = 128
C = 4
G = 64
N = B * G * G * G
NC, NS, L = 2, 16, 16
NW = NC * NS
PER_TILE = N // NW
P = 1024
NCHUNK = PER_TILE // P

_CORNERS = [(dx, dy, dz) for dx in (0, 1) for dy in (0, 1) for dz in (0, 1)]
_OFFS = [dx * S * S + dy * S + dz for (dx, dy, dz) in _CORNERS]


def _sc_body(table_hbm, cx_hbm, cy_hbm, cz_hbm, out_hbm,
             cxv, cyv, czv, fxv, fyv, fzv, idxv, rowsv, outv, sem):
    wid = lax.axis_index("s") * NC + lax.axis_index("c")
    tile_base = wid * PER_TILE
    lane = lax.iota(jnp.int32, L)

    @pl.loop(0, PER_TILE, step=P)
    def _chunk(off):
        start = tile_base + off
        pltpu.sync_copy(cx_hbm.at[pl.ds(start, P)], cxv)
        pltpu.sync_copy(cy_hbm.at[pl.ds(start, P)], cyv)
        pltpu.sync_copy(cz_hbm.at[pl.ds(start, P)], czv)

        @pl.loop(0, P, step=L)
        def _prep(i):
            x = cxv[pl.ds(i, L)]
            y = cyv[pl.ds(i, L)]
            z = czv[pl.ds(i, L)]
            ix = jnp.clip(x.astype(jnp.int32), 0, S - 2)
            iy = jnp.clip(y.astype(jnp.int32), 0, S - 2)
            iz = jnp.clip(z.astype(jnp.int32), 0, S - 2)
            fxv[pl.ds(i, L)] = jnp.clip(x - ix.astype(jnp.float32), 0.0, 1.0)
            fyv[pl.ds(i, L)] = jnp.clip(y - iy.astype(jnp.float32), 0.0, 1.0)
            fzv[pl.ds(i, L)] = jnp.clip(z - iz.astype(jnp.float32), 0.0, 1.0)
            b = (start + i + lane) >> 18
            vox = (b << 21) + (ix << 14) + (iy << 7) + iz
            for c in range(8):
                idxv[pl.ds(c * P + i, L)] = vox + _OFFS[c]

        @pl.loop(0, 8 * P, step=128)
        def _gather(k):
            pltpu.async_copy(
                table_hbm.at[idxv.at[pl.ds(k, 128)]],
                rowsv.at[pl.ds(k, 128)], sem).wait()

        @pl.loop(0, P, step=L)
        def _blend(j):
            fx = fxv[pl.ds(j, L)]
            fy = fyv[pl.ds(j, L)]
            fz = fzv[pl.ds(j, L)]
            row = j + lane
            for ch in range(C):
                col = jnp.full((L,), ch, jnp.int32)
                s = [plsc.load_gather(rowsv, [c * P + row, col])
                     for c in range(8)]
                a00 = s[0] + fz * (s[1] - s[0])
                a01 = s[2] + fz * (s[3] - s[2])
                a10 = s[4] + fz * (s[5] - s[4])
                a11 = s[6] + fz * (s[7] - s[6])
                b0 = a00 + fy * (a01 - a00)
                b1 = a10 + fy * (a11 - a10)
                plsc.store_scatter(outv, [row, col], b0 + fx * (b1 - b0))

        pltpu.sync_copy(outv, out_hbm.at[pl.ds(start, P)])


@jax.jit
def kernel(inputs, sample_coords):
    table = inputs.reshape(B * S * S * S, C)
    coords = sample_coords.reshape(N, 3)
    cx = coords[:, 0]
    cy = coords[:, 1]
    cz = coords[:, 2]

    cp = pltpu.CompilerParams(
        needs_layout_passes=False, use_tc_tiling_on_sc=False)
    mesh = plsc.VectorSubcoreMesh(core_axis_name="c", subcore_axis_name="s")
    run = pl.kernel(
        _sc_body,
        out_type=jax.ShapeDtypeStruct((N, C), jnp.float32),
        mesh=mesh,
        scratch_types=[
            pltpu.VMEM((P,), jnp.float32),
            pltpu.VMEM((P,), jnp.float32),
            pltpu.VMEM((P,), jnp.float32),
            pltpu.VMEM((P,), jnp.float32),
            pltpu.VMEM((P,), jnp.float32),
            pltpu.VMEM((P,), jnp.float32),
            pltpu.VMEM((8 * P,), jnp.int32),
            pltpu.VMEM((8 * P, C), jnp.float32),
            pltpu.VMEM((P, C), jnp.float32),
            pltpu.SemaphoreType.DMA,
        ],
        compiler_params=cp,
    )
    out = run(table, cx, cy, cz)
    return out.reshape(B, G, G, G, C)
